# Initial kernel scaffold; baseline (speedup 1.0000x reference)
import jax
import jax.numpy as jnp
from jax.experimental import pallas as pl


def kernel(q, k, v, gate_w1, gate_b1, gate_w2, gate_b2, comp_w1, comp_b1, comp_w2, comp_b2, pos_enc):
    raise NotImplementedError("write your pallas kernel here")


# speedup vs baseline: 9.1867x; 9.1867x over previous
import jax
import jax.numpy as jnp
from jax.experimental import pallas as pl

_B, _S, _H, _D = 1, 2048, 12, 64
_W = 32
_BS = 32
_STRIDE = 16
_CBS = 8
_NSEL = 4
_GH = 128
_CH = 512
_NB = (_S - _BS) // _STRIDE + 1
_NC = _NB * _CBS
_QT = 512
_NT = _S // _QT
_WT = _QT // _W
_SCALE = 0.125


def _compress_kernel(k_ref, v_ref, pe_ref, w1_ref, b1_ref, w2_ref, b2_ref,
                     kcf_ref, vcf_ref):
    k_h = k_ref[0].reshape(_S // _STRIDE, _STRIDE, _D)
    v_h = v_ref[0].reshape(_S // _STRIDE, _STRIDE, _D)
    pe = pe_ref[...]
    w1b = w1_ref[...].astype(jnp.bfloat16)
    b1 = b1_ref[...]
    w2b = w2_ref[...].astype(jnp.bfloat16)
    b2 = b2_ref[...]

    def layer1(xs, use_pe):
        acc = jnp.broadcast_to(b1, (_NB, _CH))
        for bs in range(_BS):
            j = bs % _STRIDE
            sl = xs[:, j, :]
            rows = sl[:_NB] if bs < _STRIDE else sl[1:]
            if use_pe:
                rows = rows + pe[bs:bs + 1, :]
            wseg = w1b[bs * _D:(bs + 1) * _D]
            acc = acc + jnp.dot(rows.astype(jnp.bfloat16), wseg,
                                preferred_element_type=jnp.float32)
        return jax.nn.gelu(acc)

    h_k = layer1(k_h, True)
    h_v = layer1(v_h, False)

    def layer2(h):
        hb = h.astype(jnp.bfloat16)
        parts = []
        for c in range(_CBS):
            w2c = w2b[:, c * _D:(c + 1) * _D]
            parts.append(jnp.dot(hb, w2c,
                                 preferred_element_type=jnp.float32)
                         + b2[c:c + 1, :])
        return jnp.stack(parts, axis=1).reshape(_NC, _D)

    kcf_ref[0] = layer2(h_k)
    vcf_ref[0] = layer2(h_v)


def _attn_kernel(q_ref, k_ref, ksh_ref, v_ref, vsh_ref, kcf_ref, vcf_ref,
                 gw1_ref, gb1_ref, gw2_ref, gb2_ref, o_ref):
    t = pl.program_id(1)
    q = q_ref[0]
    kcf = kcf_ref[0]
    vcf = vcf_ref[0]

    gh = jax.nn.gelu(
        jnp.dot(q, gw1_ref[...], preferred_element_type=jnp.float32, precision=jax.lax.Precision.HIGHEST)
        + gb1_ref[...])
    g = jax.nn.sigmoid(
        jnp.dot(gh, gw2_ref[...], preferred_element_type=jnp.float32, precision=jax.lax.Precision.HIGHEST)
        + gb2_ref[...])
    g = g / (jnp.sum(g, axis=1, keepdims=True) + 1e-6)

    sc = jax.lax.dot_general(
        q, kcf, (((1,), (1,)), ((), ())),
        preferred_element_type=jnp.float32, precision=jax.lax.Precision.HIGHEST) * _SCALE
    srow = jax.lax.broadcasted_iota(jnp.int32, (_QT, _NC), 0) + t * _QT
    ccol = jax.lax.broadcasted_iota(jnp.int32, (_QT, _NC), 1)
    m_cmp = srow >= (ccol // _CBS) * _STRIDE

    def softmax(x, m):
        xm = jnp.where(m, x, -1e9)
        mx = jnp.max(xm, axis=-1, keepdims=True)
        e = jnp.exp(xm - mx)
        return e / jnp.sum(e, axis=-1, keepdims=True)

    p_cmp = softmax(sc, m_cmp)
    out_cmp = jnp.dot(p_cmp, vcf, preferred_element_type=jnp.float32, precision=jax.lax.Precision.HIGHEST)

    ksum = jnp.sum(kcf.reshape(_NB, _CBS, _D), axis=1)
    bsc = jax.lax.dot_general(
        q.astype(jnp.bfloat16), ksum.astype(jnp.bfloat16),
        (((1,), (1,)), ((), ())),
        preferred_element_type=jnp.float32) * _SCALE
    brow = jax.lax.broadcasted_iota(jnp.int32, (_QT, _NB), 0) + t * _QT
    bcol = jax.lax.broadcasted_iota(jnp.int32, (_QT, _NB), 1)
    cur = jnp.where(brow >= bcol * _STRIDE, bsc, -1e9)
    ncol = ccol // _CBS
    selc = jnp.zeros((_QT, _NC), jnp.bool_)
    for _ in range(_NSEL):
        mx = jnp.max(cur, axis=1, keepdims=True)
        idx = jnp.min(jnp.where(cur >= mx, bcol, _NB), axis=1, keepdims=True)
        cur = jnp.where(bcol == idx, -1e30, cur)
        selc = selc | (ncol == idx)
    p_sel = softmax(sc, selc & m_cmp)
    out_sel = jnp.dot(p_sel, vcf, preferred_element_type=jnp.float32, precision=jax.lax.Precision.HIGHEST)

    kc = k_ref[0].reshape(_WT, _W, _D)
    kp = ksh_ref[0].reshape(_WT, _W, _D)
    vc = v_ref[0].reshape(_WT, _W, _D)
    vp = vsh_ref[0].reshape(_WT, _W, _D)
    k2 = jnp.concatenate([kp, kc], axis=1)
    v2 = jnp.concatenate([vp, vc], axis=1)
    qw = q.reshape(_WT, _W, _D)
    scw = jax.lax.dot_general(
        qw, k2, (((2,), (2,)), ((0,), (0,))),
        preferred_element_type=jnp.float32, precision=jax.lax.Precision.HIGHEST) * _SCALE
    ii = jax.lax.broadcasted_iota(jnp.int32, (_WT, _W, 2 * _W), 1)
    jj = jax.lax.broadcasted_iota(jnp.int32, (_WT, _W, 2 * _W), 2)
    uu = jax.lax.broadcasted_iota(jnp.int32, (_WT, _W, 2 * _W), 0) + t * _WT
    mw = (jj >= ii + 1) & (jj <= ii + _W) & ((uu > 0) | (jj >= _W))
    xm = jnp.where(mw, scw, -1e9)
    mxw = jnp.max(xm, axis=2, keepdims=True)
    ew = jnp.exp(xm - mxw)
    pw = ew / jnp.sum(ew, axis=2, keepdims=True)
    out_win = jax.lax.dot_general(
        pw, v2, (((2,), (1,)), ((0,), (0,))),
        preferred_element_type=jnp.float32, precision=jax.lax.Precision.HIGHEST).reshape(_QT, _D)

    o_ref[0] = (out_cmp * g[:, 0:1] + out_sel * g[:, 1:2]
                + out_win * g[:, 2:3])


def kernel(q, k, v, gate_w1, gate_b1, gate_w2, gate_b2,
           comp_w1, comp_b1, comp_w2, comp_b2, pos_enc):
    q_t = jnp.transpose(q[0], (1, 0, 2))
    k_t = jnp.transpose(k[0], (1, 0, 2))
    v_t = jnp.transpose(v[0], (1, 0, 2))
    pe = pos_enc
    cb1 = comp_b1.reshape(1, _CH)
    cb2 = comp_b2.reshape(_CBS, _D)
    gb1 = gate_b1.reshape(1, _GH)
    gb2 = gate_b2.reshape(1, 3)

    kcf, vcf = pl.pallas_call(
        _compress_kernel,
        grid=(_H,),
        in_specs=[
            pl.BlockSpec((1, _S, _D), lambda h: (h, 0, 0)),
            pl.BlockSpec((1, _S, _D), lambda h: (h, 0, 0)),
            pl.BlockSpec((_BS, _D), lambda h: (0, 0)),
            pl.BlockSpec((_BS * _D, _CH), lambda h: (0, 0)),
            pl.BlockSpec((1, _CH), lambda h: (0, 0)),
            pl.BlockSpec((_CH, _CBS * _D), lambda h: (0, 0)),
            pl.BlockSpec((_CBS, _D), lambda h: (0, 0)),
        ],
        out_specs=[
            pl.BlockSpec((1, _NC, _D), lambda h: (h, 0, 0)),
            pl.BlockSpec((1, _NC, _D), lambda h: (h, 0, 0)),
        ],
        out_shape=[
            jax.ShapeDtypeStruct((_H, _NC, _D), jnp.float32),
            jax.ShapeDtypeStruct((_H, _NC, _D), jnp.float32),
        ],
    )(k_t, v_t, pe, comp_w1, cb1, comp_w2, cb2)

    zeros = jnp.zeros((_H, _W, _D), jnp.float32)
    k_sh = jnp.concatenate([zeros, k_t[:, :-_W]], axis=1)
    v_sh = jnp.concatenate([zeros, v_t[:, :-_W]], axis=1)

    out_t = pl.pallas_call(
        _attn_kernel,
        grid=(_H, _NT),
        in_specs=[
            pl.BlockSpec((1, _QT, _D), lambda h, t: (h, t, 0)),
            pl.BlockSpec((1, _QT, _D), lambda h, t: (h, t, 0)),
            pl.BlockSpec((1, _QT, _D), lambda h, t: (h, t, 0)),
            pl.BlockSpec((1, _QT, _D), lambda h, t: (h, t, 0)),
            pl.BlockSpec((1, _QT, _D), lambda h, t: (h, t, 0)),
            pl.BlockSpec((1, _NC, _D), lambda h, t: (h, 0, 0)),
            pl.BlockSpec((1, _NC, _D), lambda h, t: (h, 0, 0)),
            pl.BlockSpec((_D, _GH), lambda h, t: (0, 0)),
            pl.BlockSpec((1, _GH), lambda h, t: (0, 0)),
            pl.BlockSpec((_GH, 3), lambda h, t: (0, 0)),
            pl.BlockSpec((1, 3), lambda h, t: (0, 0)),
        ],
        out_specs=pl.BlockSpec((1, _QT, _D), lambda h, t: (h, t, 0)),
        out_shape=jax.ShapeDtypeStruct((_H, _S, _D), jnp.float32),
    )(q_t, k_t, k_sh, v_t, v_sh, kcf, vcf, gate_w1, gb1, gate_w2, gb2)

    return jnp.transpose(out_t, (1, 0, 2))[None]


# speedup vs baseline: 17.7348x; 1.9305x over previous
import jax
import jax.numpy as jnp
from jax.experimental import pallas as pl

_B, _S, _H, _D = 1, 2048, 12, 64
_W = 32
_BS = 32
_STRIDE = 16
_CBS = 8
_NSEL = 4
_GH = 128
_CH = 512
_NB = (_S - _BS) // _STRIDE + 1
_NC = _NB * _CBS
_QT = 512
_NT = _S // _QT
_WT = _QT // _W
_SCALE = 0.125


def _compress_kernel(k_ref, v_ref, pe_ref, w1_ref, b1_ref, w2_ref, b2_ref,
                     kcf_ref, vcf_ref):
    k_h = k_ref[0].reshape(_S // _STRIDE, _STRIDE, _D)
    v_h = v_ref[0].reshape(_S // _STRIDE, _STRIDE, _D)
    pe = pe_ref[...]
    w1b = w1_ref[...].astype(jnp.bfloat16)
    b1 = b1_ref[...]
    w2b = w2_ref[...].astype(jnp.bfloat16)
    b2 = b2_ref[...]

    def layer1(xs, use_pe):
        acc = jnp.broadcast_to(b1, (_NB, _CH))
        for bs in range(_BS):
            j = bs % _STRIDE
            sl = xs[:, j, :]
            rows = sl[:_NB] if bs < _STRIDE else sl[1:]
            if use_pe:
                rows = rows + pe[bs:bs + 1, :]
            wseg = w1b[bs * _D:(bs + 1) * _D]
            acc = acc + jnp.dot(rows.astype(jnp.bfloat16), wseg,
                                preferred_element_type=jnp.float32)
        return jax.nn.gelu(acc)

    h_k = layer1(k_h, True)
    h_v = layer1(v_h, False)

    def layer2(h):
        hb = h.astype(jnp.bfloat16)
        parts = []
        for c in range(_CBS):
            w2c = w2b[:, c * _D:(c + 1) * _D]
            parts.append(jnp.dot(hb, w2c,
                                 preferred_element_type=jnp.float32)
                         + b2[c:c + 1, :])
        return jnp.stack(parts, axis=1).reshape(_NC, _D)

    kcf_ref[0] = layer2(h_k)
    vcf_ref[0] = layer2(h_v)


def _attn_kernel(q_ref, k_ref, ksh_ref, v_ref, vsh_ref, kcf_ref, vcf_ref,
                 gw1_ref, gb1_ref, gw2_ref, gb2_ref, e_ref, o_ref):
    t = pl.program_id(1)
    q = q_ref[0]
    kcf = kcf_ref[0]
    vcfb = vcf_ref[0].astype(jnp.bfloat16)
    qb = q.astype(jnp.bfloat16)
    ex = e_ref[...]

    gh = jax.nn.gelu(
        jnp.dot(qb, gw1_ref[...].astype(jnp.bfloat16),
                preferred_element_type=jnp.float32) + gb1_ref[...])
    g = jax.nn.sigmoid(
        jnp.dot(gh.astype(jnp.bfloat16), gw2_ref[...].astype(jnp.bfloat16),
                preferred_element_type=jnp.float32) + gb2_ref[...])
    g = g / (jnp.sum(g, axis=1, keepdims=True) + 1e-6)

    sc = jax.lax.dot_general(
        qb, kcf.astype(jnp.bfloat16), (((1,), (1,)), ((), ())),
        preferred_element_type=jnp.float32) * _SCALE

    ksum = jnp.sum(kcf.reshape(_NB, _CBS, _D), axis=1)
    bsc = jax.lax.dot_general(
        qb, ksum.astype(jnp.bfloat16), (((1,), (1,)), ((), ())),
        preferred_element_type=jnp.float32) * _SCALE
    brow = jax.lax.broadcasted_iota(jnp.int32, (_QT, _NB), 0) + t * _QT
    bcol = jax.lax.broadcasted_iota(jnp.int32, (_QT, _NB), 1)
    mb = brow >= bcol * _STRIDE
    cur = jnp.where(mb, bsc, -1e9)
    selb = jnp.zeros((_QT, _NB), jnp.bool_)
    for _ in range(_NSEL):
        mxb = jnp.max(cur, axis=1, keepdims=True)
        idx = jnp.min(jnp.where(cur >= mxb, bcol, _NB), axis=1, keepdims=True)
        onehot = bcol == idx
        selb = selb | onehot
        cur = jnp.where(onehot, -1e30, cur)

    mbf = jnp.where(mb, 1.0, 0.0).astype(jnp.bfloat16)
    sbf = jnp.where(selb & mb, 1.0, 0.0).astype(jnp.bfloat16)
    m01 = jnp.dot(mbf, ex, preferred_element_type=jnp.float32)
    s01 = jnp.dot(sbf, ex, preferred_element_type=jnp.float32)

    mx = jnp.max(sc, axis=1, keepdims=True)
    e = jnp.exp(sc - mx)
    me = e * m01
    mesel = e * s01
    s_cmp = jnp.sum(me, axis=1, keepdims=True)
    s_sel = jnp.sum(mesel, axis=1, keepdims=True)
    o_cmp = jnp.dot(me.astype(jnp.bfloat16), vcfb,
                    preferred_element_type=jnp.float32)
    o_sel = jnp.dot(mesel.astype(jnp.bfloat16), vcfb,
                    preferred_element_type=jnp.float32)

    kc = k_ref[0].reshape(_WT, _W, _D)
    kp = ksh_ref[0].reshape(_WT, _W, _D)
    vc = v_ref[0].reshape(_WT, _W, _D)
    vp = vsh_ref[0].reshape(_WT, _W, _D)
    k2 = jnp.concatenate([kp, kc], axis=1).astype(jnp.bfloat16)
    v2 = jnp.concatenate([vp, vc], axis=1).astype(jnp.bfloat16)
    qw = qb.reshape(_WT, _W, _D)
    scw = jax.lax.dot_general(
        qw, k2, (((2,), (2,)), ((0,), (0,))),
        preferred_element_type=jnp.float32) * _SCALE
    ii = jax.lax.broadcasted_iota(jnp.int32, (_WT, _W, 2 * _W), 1)
    jj = jax.lax.broadcasted_iota(jnp.int32, (_WT, _W, 2 * _W), 2)
    uu = jax.lax.broadcasted_iota(jnp.int32, (_WT, _W, 2 * _W), 0) + t * _WT
    mw = (jj >= ii + 1) & (jj <= ii + _W) & ((uu > 0) | (jj >= _W))
    xm = jnp.where(mw, scw, -1e9)
    mxw = jnp.max(xm, axis=2, keepdims=True)
    ew = jnp.exp(xm - mxw)
    pw = ew / jnp.sum(ew, axis=2, keepdims=True)
    out_win = jax.lax.dot_general(
        pw.astype(jnp.bfloat16), v2, (((2,), (1,)), ((0,), (0,))),
        preferred_element_type=jnp.float32).reshape(_QT, _D)

    o_ref[0] = (o_cmp * (g[:, 0:1] / s_cmp) + o_sel * (g[:, 1:2] / s_sel)
                + out_win * g[:, 2:3])


def kernel(q, k, v, gate_w1, gate_b1, gate_w2, gate_b2,
           comp_w1, comp_b1, comp_w2, comp_b2, pos_enc):
    q_t = jnp.transpose(q[0], (1, 0, 2))
    k_t = jnp.transpose(k[0], (1, 0, 2))
    v_t = jnp.transpose(v[0], (1, 0, 2))
    pe = pos_enc
    cb1 = comp_b1.reshape(1, _CH)
    cb2 = comp_b2.reshape(_CBS, _D)
    gb1 = gate_b1.reshape(1, _GH)
    gb2 = gate_b2.reshape(1, 3)

    kcf, vcf = pl.pallas_call(
        _compress_kernel,
        grid=(_H,),
        in_specs=[
            pl.BlockSpec((1, _S, _D), lambda h: (h, 0, 0)),
            pl.BlockSpec((1, _S, _D), lambda h: (h, 0, 0)),
            pl.BlockSpec((_BS, _D), lambda h: (0, 0)),
            pl.BlockSpec((_BS * _D, _CH), lambda h: (0, 0)),
            pl.BlockSpec((1, _CH), lambda h: (0, 0)),
            pl.BlockSpec((_CH, _CBS * _D), lambda h: (0, 0)),
            pl.BlockSpec((_CBS, _D), lambda h: (0, 0)),
        ],
        out_specs=[
            pl.BlockSpec((1, _NC, _D), lambda h: (h, 0, 0)),
            pl.BlockSpec((1, _NC, _D), lambda h: (h, 0, 0)),
        ],
        out_shape=[
            jax.ShapeDtypeStruct((_H, _NC, _D), jnp.float32),
            jax.ShapeDtypeStruct((_H, _NC, _D), jnp.float32),
        ],
    )(k_t, v_t, pe, comp_w1, cb1, comp_w2, cb2)

    zeros = jnp.zeros((_H, _W, _D), jnp.float32)
    k_sh = jnp.concatenate([zeros, k_t[:, :-_W]], axis=1)
    v_sh = jnp.concatenate([zeros, v_t[:, :-_W]], axis=1)
    expand = jnp.repeat(jnp.eye(_NB, dtype=jnp.bfloat16), _CBS, axis=1)

    out_t = pl.pallas_call(
        _attn_kernel,
        grid=(_H, _NT),
        in_specs=[
            pl.BlockSpec((1, _QT, _D), lambda h, t: (h, t, 0)),
            pl.BlockSpec((1, _QT, _D), lambda h, t: (h, t, 0)),
            pl.BlockSpec((1, _QT, _D), lambda h, t: (h, t, 0)),
            pl.BlockSpec((1, _QT, _D), lambda h, t: (h, t, 0)),
            pl.BlockSpec((1, _QT, _D), lambda h, t: (h, t, 0)),
            pl.BlockSpec((1, _NC, _D), lambda h, t: (h, 0, 0)),
            pl.BlockSpec((1, _NC, _D), lambda h, t: (h, 0, 0)),
            pl.BlockSpec((_D, _GH), lambda h, t: (0, 0)),
            pl.BlockSpec((1, _GH), lambda h, t: (0, 0)),
            pl.BlockSpec((_GH, 3), lambda h, t: (0, 0)),
            pl.BlockSpec((1, 3), lambda h, t: (0, 0)),
            pl.BlockSpec((_NB, _NC), lambda h, t: (0, 0)),
        ],
        out_specs=pl.BlockSpec((1, _QT, _D), lambda h, t: (h, t, 0)),
        out_shape=jax.ShapeDtypeStruct((_H, _S, _D), jnp.float32),
    )(q_t, k_t, k_sh, v_t, v_sh, kcf, vcf, gate_w1, gb1, gate_w2, gb2, expand)

    return jnp.transpose(out_t, (1, 0, 2))[None]


# speedup vs baseline: 18.4953x; 1.0429x over previous
import jax
import jax.numpy as jnp
from jax.experimental import pallas as pl

_B, _S, _H, _D = 1, 2048, 12, 64
_W = 32
_BS = 32
_STRIDE = 16
_CBS = 8
_NSEL = 4
_GH = 128
_CH = 512
_NB = (_S - _BS) // _STRIDE + 1
_NC = _NB * _CBS
_QT = 512
_NT = _S // _QT
_WT = _QT // _W
_SCALE = 0.125


def _compress_kernel(k_ref, v_ref, pe_ref, w1_ref, b1_ref, w2_ref, b2_ref,
                     kcf_ref, vcf_ref):
    k_h = k_ref[0].reshape(_S // _STRIDE, _STRIDE, _D)
    v_h = v_ref[0].reshape(_S // _STRIDE, _STRIDE, _D)
    pe = pe_ref[...]
    w1b = w1_ref[...].astype(jnp.bfloat16)
    b1 = b1_ref[...]
    w2b = w2_ref[...].astype(jnp.bfloat16)
    b2 = b2_ref[...]

    def layer1(xs, use_pe):
        acc = jnp.broadcast_to(b1, (_NB, _CH))
        for bs in range(_BS):
            j = bs % _STRIDE
            sl = xs[:, j, :]
            rows = sl[:_NB] if bs < _STRIDE else sl[1:]
            if use_pe:
                rows = rows + pe[bs:bs + 1, :]
            wseg = w1b[bs * _D:(bs + 1) * _D]
            acc = acc + jnp.dot(rows.astype(jnp.bfloat16), wseg,
                                preferred_element_type=jnp.float32)
        return jax.nn.gelu(acc)

    h_k = layer1(k_h, True)
    h_v = layer1(v_h, False)

    def layer2(h):
        hb = h.astype(jnp.bfloat16)
        parts = []
        for c in range(_CBS):
            w2c = w2b[:, c * _D:(c + 1) * _D]
            parts.append(jnp.dot(hb, w2c,
                                 preferred_element_type=jnp.float32)
                         + b2[c:c + 1, :])
        return jnp.stack(parts, axis=1).reshape(_NC, _D)

    kcf_ref[0] = layer2(h_k)
    vcf_ref[0] = layer2(h_v)


def _attn_kernel(q_ref, k_ref, ksh_ref, v_ref, vsh_ref, kcf_ref, vcf_ref,
                 gw1_ref, gb1_ref, gw2_ref, gb2_ref, e_ref, o_ref):
    t = pl.program_id(1)
    q = q_ref[0]
    kcf = kcf_ref[0]
    vcfb = vcf_ref[0].astype(jnp.bfloat16)
    vx = jnp.concatenate(
        [vcfb, jnp.ones((_NC, 1), jnp.bfloat16)], axis=1)
    qb = q.astype(jnp.bfloat16)
    qs = (q * _SCALE).astype(jnp.bfloat16)
    ex = e_ref[...]

    gh = jax.nn.gelu(
        jnp.dot(qb, gw1_ref[...].astype(jnp.bfloat16),
                preferred_element_type=jnp.float32) + gb1_ref[...])
    g = jax.nn.sigmoid(
        jnp.dot(gh.astype(jnp.bfloat16), gw2_ref[...].astype(jnp.bfloat16),
                preferred_element_type=jnp.float32) + gb2_ref[...])
    g = g / (jnp.sum(g, axis=1, keepdims=True) + 1e-6)

    sc = jax.lax.dot_general(
        qs, kcf.astype(jnp.bfloat16), (((1,), (1,)), ((), ())),
        preferred_element_type=jnp.float32)

    ksum = jnp.sum(kcf.reshape(_NB, _CBS, _D), axis=1)
    bsc = jax.lax.dot_general(
        qb, ksum.astype(jnp.bfloat16), (((1,), (1,)), ((), ())),
        preferred_element_type=jnp.float32) * _SCALE
    brow = jax.lax.broadcasted_iota(jnp.int32, (_QT, _NB), 0) + t * _QT
    bcol = jax.lax.broadcasted_iota(jnp.int32, (_QT, _NB), 1)
    mb = brow >= bcol * _STRIDE
    cur = jnp.where(mb, bsc, -1e9)
    selb = jnp.zeros((_QT, _NB), jnp.bool_)
    for _ in range(_NSEL):
        mxb = jnp.max(cur, axis=1, keepdims=True)
        idx = jnp.min(jnp.where(cur >= mxb, bcol, _NB), axis=1, keepdims=True)
        onehot = bcol == idx
        selb = selb | onehot
        cur = jnp.where(onehot, -1e30, cur)

    mbf = jnp.where(mb, 1.0, 0.0).astype(jnp.bfloat16)
    sbf = jnp.where(selb & mb, 1.0, 0.0).astype(jnp.bfloat16)
    m01 = jnp.dot(mbf, ex, preferred_element_type=jnp.float32)
    s01 = jnp.dot(sbf, ex, preferred_element_type=jnp.float32)

    mx = jnp.max(sc, axis=1, keepdims=True)
    e = jnp.exp(sc - mx)
    me = e * m01
    mesel = e * s01
    o_cmp = jnp.dot(me.astype(jnp.bfloat16), vx,
                    preferred_element_type=jnp.float32)
    o_sel = jnp.dot(mesel.astype(jnp.bfloat16), vx,
                    preferred_element_type=jnp.float32)
    s_cmp = o_cmp[:, _D:_D + 1]
    s_sel = o_sel[:, _D:_D + 1]

    kc = k_ref[0].reshape(_WT, _W, _D)
    kp = ksh_ref[0].reshape(_WT, _W, _D)
    vc = v_ref[0].reshape(_WT, _W, _D)
    vp = vsh_ref[0].reshape(_WT, _W, _D)
    k2 = jnp.concatenate([kp, kc], axis=1).astype(jnp.bfloat16)
    v2 = jnp.concatenate([vp, vc], axis=1).astype(jnp.bfloat16)
    qw = qs.reshape(_WT, _W, _D)
    scw = jax.lax.dot_general(
        qw, k2, (((2,), (2,)), ((0,), (0,))),
        preferred_element_type=jnp.float32)
    ii = jax.lax.broadcasted_iota(jnp.int32, (_WT, _W, 2 * _W), 1)
    jj = jax.lax.broadcasted_iota(jnp.int32, (_WT, _W, 2 * _W), 2)
    uu = jax.lax.broadcasted_iota(jnp.int32, (_WT, _W, 2 * _W), 0) + t * _WT
    mw = (jj >= ii + 1) & (jj <= ii + _W) & ((uu > 0) | (jj >= _W))
    xm = jnp.where(mw, scw, -1e9)
    mxw = jnp.max(xm, axis=2, keepdims=True)
    ew = jnp.exp(xm - mxw)
    pw = ew / jnp.sum(ew, axis=2, keepdims=True)
    out_win = jax.lax.dot_general(
        pw.astype(jnp.bfloat16), v2, (((2,), (1,)), ((0,), (0,))),
        preferred_element_type=jnp.float32).reshape(_QT, _D)

    o_ref[0] = (o_cmp[:, :_D] * (g[:, 0:1] / s_cmp)
                + o_sel[:, :_D] * (g[:, 1:2] / s_sel)
                + out_win * g[:, 2:3])


def kernel(q, k, v, gate_w1, gate_b1, gate_w2, gate_b2,
           comp_w1, comp_b1, comp_w2, comp_b2, pos_enc):
    q_t = jnp.transpose(q[0], (1, 0, 2))
    k_t = jnp.transpose(k[0], (1, 0, 2))
    v_t = jnp.transpose(v[0], (1, 0, 2))
    pe = pos_enc
    cb1 = comp_b1.reshape(1, _CH)
    cb2 = comp_b2.reshape(_CBS, _D)
    gb1 = gate_b1.reshape(1, _GH)
    gb2 = gate_b2.reshape(1, 3)

    kcf, vcf = pl.pallas_call(
        _compress_kernel,
        grid=(_H,),
        in_specs=[
            pl.BlockSpec((1, _S, _D), lambda h: (h, 0, 0)),
            pl.BlockSpec((1, _S, _D), lambda h: (h, 0, 0)),
            pl.BlockSpec((_BS, _D), lambda h: (0, 0)),
            pl.BlockSpec((_BS * _D, _CH), lambda h: (0, 0)),
            pl.BlockSpec((1, _CH), lambda h: (0, 0)),
            pl.BlockSpec((_CH, _CBS * _D), lambda h: (0, 0)),
            pl.BlockSpec((_CBS, _D), lambda h: (0, 0)),
        ],
        out_specs=[
            pl.BlockSpec((1, _NC, _D), lambda h: (h, 0, 0)),
            pl.BlockSpec((1, _NC, _D), lambda h: (h, 0, 0)),
        ],
        out_shape=[
            jax.ShapeDtypeStruct((_H, _NC, _D), jnp.float32),
            jax.ShapeDtypeStruct((_H, _NC, _D), jnp.float32),
        ],
    )(k_t, v_t, pe, comp_w1, cb1, comp_w2, cb2)

    zeros = jnp.zeros((_H, _W, _D), jnp.float32)
    k_sh = jnp.concatenate([zeros, k_t[:, :-_W]], axis=1)
    v_sh = jnp.concatenate([zeros, v_t[:, :-_W]], axis=1)
    expand = jnp.repeat(jnp.eye(_NB, dtype=jnp.bfloat16), _CBS, axis=1)

    out_t = pl.pallas_call(
        _attn_kernel,
        grid=(_H, _NT),
        in_specs=[
            pl.BlockSpec((1, _QT, _D), lambda h, t: (h, t, 0)),
            pl.BlockSpec((1, _QT, _D), lambda h, t: (h, t, 0)),
            pl.BlockSpec((1, _QT, _D), lambda h, t: (h, t, 0)),
            pl.BlockSpec((1, _QT, _D), lambda h, t: (h, t, 0)),
            pl.BlockSpec((1, _QT, _D), lambda h, t: (h, t, 0)),
            pl.BlockSpec((1, _NC, _D), lambda h, t: (h, 0, 0)),
            pl.BlockSpec((1, _NC, _D), lambda h, t: (h, 0, 0)),
            pl.BlockSpec((_D, _GH), lambda h, t: (0, 0)),
            pl.BlockSpec((1, _GH), lambda h, t: (0, 0)),
            pl.BlockSpec((_GH, 3), lambda h, t: (0, 0)),
            pl.BlockSpec((1, 3), lambda h, t: (0, 0)),
            pl.BlockSpec((_NB, _NC), lambda h, t: (0, 0)),
        ],
        out_specs=pl.BlockSpec((1, _QT, _D), lambda h, t: (h, t, 0)),
        out_shape=jax.ShapeDtypeStruct((_H, _S, _D), jnp.float32),
    )(q_t, k_t, k_sh, v_t, v_sh, kcf, vcf, gate_w1, gb1, gate_w2, gb2, expand)

    return jnp.transpose(out_t, (1, 0, 2))[None]


# speedup vs baseline: 22.7351x; 1.2292x over previous
import jax
import jax.numpy as jnp
from jax.experimental import pallas as pl

_B, _S, _H, _D = 1, 2048, 12, 64
_W = 32
_BS = 32
_STRIDE = 16
_CBS = 8
_NSEL = 4
_GH = 128
_CH = 512
_NB = (_S - _BS) // _STRIDE + 1
_NC = _NB * _CBS
_QT = 512
_NT = _S // _QT
_WT = _QT // _W
_SCALE = 0.125


def _compress_kernel(k16_ref, v16_ref, pe_ref, w1_ref, b1_ref, w2_ref, b2_ref,
                     kc_ref, vc_ref):
    k16 = k16_ref[0]
    v16 = v16_ref[0]
    pe = pe_ref[...]
    w1 = w1_ref[...]
    w1a = w1[:_STRIDE * _D].astype(jnp.bfloat16)
    w1b = w1[_STRIDE * _D:].astype(jnp.bfloat16)
    b1 = b1_ref[...]
    w2b = w2_ref[...].astype(jnp.bfloat16)
    b2 = b2_ref[...]

    def mlp(x16, use_pe):
        a = x16[:_NB]
        b = x16[1:]
        if use_pe:
            a = a + pe[:, :_STRIDE * _D]
            b = b + pe[:, _STRIDE * _D:]
        h = jax.nn.gelu(
            jnp.dot(a.astype(jnp.bfloat16), w1a,
                    preferred_element_type=jnp.float32)
            + jnp.dot(b.astype(jnp.bfloat16), w1b,
                      preferred_element_type=jnp.float32) + b1)
        return jnp.dot(h.astype(jnp.bfloat16), w2b,
                       preferred_element_type=jnp.float32) + b2

    kc_ref[0] = mlp(k16, True)
    vc_ref[0] = mlp(v16, False)


def _attn_kernel(q_ref, k_ref, v_ref, kcf_ref, vcf_ref,
                 gw1_ref, gb1_ref, gw2_ref, gb2_ref, e_ref, o_ref):
    t = pl.program_id(1)
    q = q_ref[0]
    kcf = kcf_ref[0]
    vcfb = vcf_ref[0].astype(jnp.bfloat16)
    vx = jnp.concatenate(
        [vcfb, jnp.ones((_NC, 1), jnp.bfloat16)], axis=1)
    qb = q.astype(jnp.bfloat16)
    qs = (q * _SCALE).astype(jnp.bfloat16)
    ex = e_ref[...]

    gh = jax.nn.gelu(
        jnp.dot(qb, gw1_ref[...].astype(jnp.bfloat16),
                preferred_element_type=jnp.float32) + gb1_ref[...])
    g = jax.nn.sigmoid(
        jnp.dot(gh.astype(jnp.bfloat16), gw2_ref[...].astype(jnp.bfloat16),
                preferred_element_type=jnp.float32) + gb2_ref[...])
    g = g / (jnp.sum(g, axis=1, keepdims=True) + 1e-6)

    sc = jax.lax.dot_general(
        qs, kcf.astype(jnp.bfloat16), (((1,), (1,)), ((), ())),
        preferred_element_type=jnp.float32)

    ksum = jnp.sum(kcf.reshape(_NB, _CBS, _D), axis=1)
    bsc = jax.lax.dot_general(
        qb, ksum.astype(jnp.bfloat16), (((1,), (1,)), ((), ())),
        preferred_element_type=jnp.float32) * _SCALE
    brow = jax.lax.broadcasted_iota(jnp.int32, (_QT, _NB), 0) + t * _QT
    bcol = jax.lax.broadcasted_iota(jnp.int32, (_QT, _NB), 1)
    mb = brow >= bcol * _STRIDE
    cur = jnp.where(mb, bsc, -1e9)
    selb = jnp.zeros((_QT, _NB), jnp.bool_)
    for _ in range(_NSEL):
        mxb = jnp.max(cur, axis=1, keepdims=True)
        idx = jnp.min(jnp.where(cur >= mxb, bcol, _NB), axis=1, keepdims=True)
        onehot = bcol == idx
        selb = selb | onehot
        cur = jnp.where(onehot, -1e30, cur)

    mbf = jnp.where(mb, 1.0, 0.0).astype(jnp.bfloat16)
    sbf = jnp.where(selb & mb, 1.0, 0.0).astype(jnp.bfloat16)
    m01 = jnp.dot(mbf, ex, preferred_element_type=jnp.float32)
    s01 = jnp.dot(sbf, ex, preferred_element_type=jnp.float32)

    mx = jnp.max(sc, axis=1, keepdims=True)
    e = jnp.exp(sc - mx)
    me = e * m01
    mesel = e * s01
    o_cmp = jnp.dot(me.astype(jnp.bfloat16), vx,
                    preferred_element_type=jnp.float32)
    o_sel = jnp.dot(mesel.astype(jnp.bfloat16), vx,
                    preferred_element_type=jnp.float32)
    s_cmp = o_cmp[:, _D:_D + 1]
    s_sel = o_sel[:, _D:_D + 1]

    pstart = jnp.maximum(_QT * t - _W, 0)

    def window_tiles(x_ref):
        cur = x_ref[0, pl.ds(_QT * t, _QT), :].reshape(_WT, _W, _D)
        prev = x_ref[0, pl.ds(pstart, _QT), :].reshape(_WT, _W, _D)
        prev0 = jnp.concatenate([prev[:1], prev[:_WT - 1]], axis=0)
        return cur, jnp.where(t == 0, prev0, prev)

    kc, kp = window_tiles(k_ref)
    vc, vp = window_tiles(v_ref)
    k2 = jnp.concatenate([kp, kc], axis=1).astype(jnp.bfloat16)
    v2 = jnp.concatenate([vp, vc], axis=1).astype(jnp.bfloat16)
    qw = qs.reshape(_WT, _W, _D)
    scw = jax.lax.dot_general(
        qw, k2, (((2,), (2,)), ((0,), (0,))),
        preferred_element_type=jnp.float32)
    ii = jax.lax.broadcasted_iota(jnp.int32, (_WT, _W, 2 * _W), 1)
    jj = jax.lax.broadcasted_iota(jnp.int32, (_WT, _W, 2 * _W), 2)
    uu = jax.lax.broadcasted_iota(jnp.int32, (_WT, _W, 2 * _W), 0) + t * _WT
    mw = (jj >= ii + 1) & (jj <= ii + _W) & ((uu > 0) | (jj >= _W))
    xm = jnp.where(mw, scw, -1e9)
    mxw = jnp.max(xm, axis=2, keepdims=True)
    ew = jnp.exp(xm - mxw)
    pw = ew / jnp.sum(ew, axis=2, keepdims=True)
    out_win = jax.lax.dot_general(
        pw.astype(jnp.bfloat16), v2, (((2,), (1,)), ((0,), (0,))),
        preferred_element_type=jnp.float32).reshape(_QT, _D)

    o_ref[0] = (o_cmp[:, :_D] * (g[:, 0:1] / s_cmp)
                + o_sel[:, :_D] * (g[:, 1:2] / s_sel)
                + out_win * g[:, 2:3])


def kernel(q, k, v, gate_w1, gate_b1, gate_w2, gate_b2,
           comp_w1, comp_b1, comp_w2, comp_b2, pos_enc):
    q_t = jnp.transpose(q[0], (1, 0, 2))
    k_t = jnp.transpose(k[0], (1, 0, 2))
    v_t = jnp.transpose(v[0], (1, 0, 2))
    k16 = k_t.reshape(_H, _S // _STRIDE, _STRIDE * _D)
    v16 = v_t.reshape(_H, _S // _STRIDE, _STRIDE * _D)
    pe = pos_enc.reshape(1, _BS * _D)
    cb1 = comp_b1.reshape(1, _CH)
    cb2 = comp_b2.reshape(1, _CBS * _D)
    gb1 = gate_b1.reshape(1, _GH)
    gb2 = gate_b2.reshape(1, 3)

    kc, vc = pl.pallas_call(
        _compress_kernel,
        grid=(_H,),
        in_specs=[
            pl.BlockSpec((1, _S // _STRIDE, _STRIDE * _D), lambda h: (h, 0, 0)),
            pl.BlockSpec((1, _S // _STRIDE, _STRIDE * _D), lambda h: (h, 0, 0)),
            pl.BlockSpec((1, _BS * _D), lambda h: (0, 0)),
            pl.BlockSpec((_BS * _D, _CH), lambda h: (0, 0)),
            pl.BlockSpec((1, _CH), lambda h: (0, 0)),
            pl.BlockSpec((_CH, _CBS * _D), lambda h: (0, 0)),
            pl.BlockSpec((1, _CBS * _D), lambda h: (0, 0)),
        ],
        out_specs=[
            pl.BlockSpec((1, _NB, _CBS * _D), lambda h: (h, 0, 0)),
            pl.BlockSpec((1, _NB, _CBS * _D), lambda h: (h, 0, 0)),
        ],
        out_shape=[
            jax.ShapeDtypeStruct((_H, _NB, _CBS * _D), jnp.float32),
            jax.ShapeDtypeStruct((_H, _NB, _CBS * _D), jnp.float32),
        ],
    )(k16, v16, pe, comp_w1, cb1, comp_w2, cb2)

    kcf = kc.reshape(_H, _NC, _D)
    vcf = vc.reshape(_H, _NC, _D)
    expand = jnp.repeat(jnp.eye(_NB, dtype=jnp.bfloat16), _CBS, axis=1)

    out = pl.pallas_call(
        _attn_kernel,
        grid=(_H, _NT),
        in_specs=[
            pl.BlockSpec((1, _QT, _D), lambda h, t: (h, t, 0)),
            pl.BlockSpec((1, _S, _D), lambda h, t: (h, 0, 0)),
            pl.BlockSpec((1, _S, _D), lambda h, t: (h, 0, 0)),
            pl.BlockSpec((1, _NC, _D), lambda h, t: (h, 0, 0)),
            pl.BlockSpec((1, _NC, _D), lambda h, t: (h, 0, 0)),
            pl.BlockSpec((_D, _GH), lambda h, t: (0, 0)),
            pl.BlockSpec((1, _GH), lambda h, t: (0, 0)),
            pl.BlockSpec((_GH, 3), lambda h, t: (0, 0)),
            pl.BlockSpec((1, 3), lambda h, t: (0, 0)),
            pl.BlockSpec((_NB, _NC), lambda h, t: (0, 0)),
        ],
        out_specs=pl.BlockSpec((1, _QT, _D), lambda h, t: (h, t, 0)),
        out_shape=jax.ShapeDtypeStruct((_H, _S, _D), jnp.float32),
    )(q_t, k_t, v_t, kcf, vcf, gate_w1, gb1, gate_w2, gb2, expand)

    return jnp.transpose(out, (1, 0, 2))[None]


# speedup vs baseline: 25.6941x; 1.1301x over previous
import jax
import jax.numpy as jnp
from jax.experimental import pallas as pl

_B, _S, _H, _D = 1, 2048, 12, 64
_W = 32
_BS = 32
_STRIDE = 16
_CBS = 8
_NSEL = 4
_GH = 128
_CH = 512
_NB = (_S - _BS) // _STRIDE + 1
_NC = _NB * _CBS
_QT = 512
_NT = _S // _QT
_WT = _QT // _W
_SCALE = 0.125


def _compress_kernel(k16_ref, v16_ref, pe_ref, w1_ref, b1_ref, w2_ref, b2_ref,
                     m_ref, kc_ref, vc_ref, ks_ref):
    k16 = k16_ref[0]
    v16 = v16_ref[0]
    pe = pe_ref[...]
    w1 = w1_ref[...]
    w1a = w1[:_STRIDE * _D].astype(jnp.bfloat16)
    w1b = w1[_STRIDE * _D:].astype(jnp.bfloat16)
    b1 = b1_ref[...]
    w2b = w2_ref[...].astype(jnp.bfloat16)
    b2 = b2_ref[...]

    def mlp(x16, use_pe):
        a = x16[:_NB]
        b = x16[1:]
        if use_pe:
            a = a + pe[:, :_STRIDE * _D]
            b = b + pe[:, _STRIDE * _D:]
        h = jax.nn.gelu(
            jnp.dot(a.astype(jnp.bfloat16), w1a,
                    preferred_element_type=jnp.float32)
            + jnp.dot(b.astype(jnp.bfloat16), w1b,
                      preferred_element_type=jnp.float32) + b1)
        return jnp.dot(h.astype(jnp.bfloat16), w2b,
                       preferred_element_type=jnp.float32) + b2

    kcmp = mlp(k16, True)
    kc_ref[0] = kcmp
    vc_ref[0] = mlp(v16, False)
    ks_ref[0] = jnp.dot(kcmp, m_ref[...],
                        preferred_element_type=jnp.float32,
                        precision=jax.lax.Precision.HIGHEST)


def _attn_kernel(q_ref, k_ref, v_ref, kcf_ref, vcf_ref, ks_ref,
                 gw1_ref, gb1_ref, gw2_ref, gb2_ref, e_ref, o_ref):
    t = pl.program_id(1)
    q = q_ref[0]
    kcf = kcf_ref[0]
    vcfb = vcf_ref[0].astype(jnp.bfloat16)
    vx = jnp.concatenate(
        [vcfb, jnp.ones((_NC, 1), jnp.bfloat16)], axis=1)
    qb = q.astype(jnp.bfloat16)
    qs = (q * _SCALE).astype(jnp.bfloat16)
    ex = e_ref[...]

    gh = jax.nn.gelu(
        jnp.dot(qb, gw1_ref[...].astype(jnp.bfloat16),
                preferred_element_type=jnp.float32) + gb1_ref[...])
    g = jax.nn.sigmoid(
        jnp.dot(gh.astype(jnp.bfloat16), gw2_ref[...].astype(jnp.bfloat16),
                preferred_element_type=jnp.float32) + gb2_ref[...])
    g = g / (jnp.sum(g, axis=1, keepdims=True) + 1e-6)

    sc = jax.lax.dot_general(
        qs, kcf.astype(jnp.bfloat16), (((1,), (1,)), ((), ())),
        preferred_element_type=jnp.float32)

    bsc = jax.lax.dot_general(
        qb, ks_ref[0].astype(jnp.bfloat16), (((1,), (1,)), ((), ())),
        preferred_element_type=jnp.float32) * _SCALE
    brow = jax.lax.broadcasted_iota(jnp.int32, (_QT, _NB), 0) + t * _QT
    bcol = jax.lax.broadcasted_iota(jnp.int32, (_QT, _NB), 1)
    mb = brow >= bcol * _STRIDE
    cm = jnp.where(mb, bsc, -1e9)
    cur = cm
    for _ in range(_NSEL - 1):
        mxb = jnp.max(cur, axis=1, keepdims=True)
        cur = jnp.where(cur >= mxb, -3e38, cur)
    thr = jnp.max(cur, axis=1, keepdims=True)
    mbf = jnp.where(mb, 1.0, 0.0).astype(jnp.bfloat16)
    sbf = jnp.where((cm >= thr) & mb, 1.0, 0.0).astype(jnp.bfloat16)
    m01 = jnp.dot(mbf, ex, preferred_element_type=jnp.float32)
    s01 = jnp.dot(sbf, ex, preferred_element_type=jnp.float32)

    mx = jnp.max(sc, axis=1, keepdims=True)
    e = jnp.exp(sc - mx)
    me = e * m01
    mesel = e * s01
    o_cmp = jnp.dot(me.astype(jnp.bfloat16), vx,
                    preferred_element_type=jnp.float32)
    o_sel = jnp.dot(mesel.astype(jnp.bfloat16), vx,
                    preferred_element_type=jnp.float32)
    s_cmp = o_cmp[:, _D:_D + 1]
    s_sel = o_sel[:, _D:_D + 1]

    pstart = jnp.maximum(_QT * t - _W, 0)

    def window_tiles(x_ref):
        cur = x_ref[0, pl.ds(_QT * t, _QT), :].reshape(_WT, _W, _D)
        prev = x_ref[0, pl.ds(pstart, _QT), :].reshape(_WT, _W, _D)
        prev0 = jnp.concatenate([prev[:1], prev[:_WT - 1]], axis=0)
        return cur, jnp.where(t == 0, prev0, prev)

    kc, kp = window_tiles(k_ref)
    vc, vp = window_tiles(v_ref)
    k2 = jnp.concatenate([kp, kc], axis=1).astype(jnp.bfloat16)
    v2 = jnp.concatenate([vp, vc], axis=1).astype(jnp.bfloat16)
    qw = qs.reshape(_WT, _W, _D)
    scw = jax.lax.dot_general(
        qw, k2, (((2,), (2,)), ((0,), (0,))),
        preferred_element_type=jnp.float32)
    ii = jax.lax.broadcasted_iota(jnp.int32, (_WT, _W, 2 * _W), 1)
    jj = jax.lax.broadcasted_iota(jnp.int32, (_WT, _W, 2 * _W), 2)
    uu = jax.lax.broadcasted_iota(jnp.int32, (_WT, _W, 2 * _W), 0) + t * _WT
    mw = (jj >= ii + 1) & (jj <= ii + _W) & ((uu > 0) | (jj >= _W))
    xm = jnp.where(mw, scw, -1e9)
    mxw = jnp.max(xm, axis=2, keepdims=True)
    ew = jnp.exp(xm - mxw)
    pw = ew / jnp.sum(ew, axis=2, keepdims=True)
    out_win = jax.lax.dot_general(
        pw.astype(jnp.bfloat16), v2, (((2,), (1,)), ((0,), (0,))),
        preferred_element_type=jnp.float32).reshape(_QT, _D)

    o_ref[0] = (o_cmp[:, :_D] * (g[:, 0:1] / s_cmp)
                + o_sel[:, :_D] * (g[:, 1:2] / s_sel)
                + out_win * g[:, 2:3])


def kernel(q, k, v, gate_w1, gate_b1, gate_w2, gate_b2,
           comp_w1, comp_b1, comp_w2, comp_b2, pos_enc):
    q_t = jnp.transpose(q[0], (1, 0, 2))
    k_t = jnp.transpose(k[0], (1, 0, 2))
    v_t = jnp.transpose(v[0], (1, 0, 2))
    k16 = k_t.reshape(_H, _S // _STRIDE, _STRIDE * _D)
    v16 = v_t.reshape(_H, _S // _STRIDE, _STRIDE * _D)
    pe = pos_enc.reshape(1, _BS * _D)
    cb1 = comp_b1.reshape(1, _CH)
    cb2 = comp_b2.reshape(1, _CBS * _D)
    gb1 = gate_b1.reshape(1, _GH)
    gb2 = gate_b2.reshape(1, 3)

    csum = jnp.tile(jnp.eye(_D, dtype=jnp.float32), (_CBS, 1))

    kc, vc, ks = pl.pallas_call(
        _compress_kernel,
        grid=(_H,),
        in_specs=[
            pl.BlockSpec((1, _S // _STRIDE, _STRIDE * _D), lambda h: (h, 0, 0)),
            pl.BlockSpec((1, _S // _STRIDE, _STRIDE * _D), lambda h: (h, 0, 0)),
            pl.BlockSpec((1, _BS * _D), lambda h: (0, 0)),
            pl.BlockSpec((_BS * _D, _CH), lambda h: (0, 0)),
            pl.BlockSpec((1, _CH), lambda h: (0, 0)),
            pl.BlockSpec((_CH, _CBS * _D), lambda h: (0, 0)),
            pl.BlockSpec((1, _CBS * _D), lambda h: (0, 0)),
            pl.BlockSpec((_CBS * _D, _D), lambda h: (0, 0)),
        ],
        out_specs=[
            pl.BlockSpec((1, _NB, _CBS * _D), lambda h: (h, 0, 0)),
            pl.BlockSpec((1, _NB, _CBS * _D), lambda h: (h, 0, 0)),
            pl.BlockSpec((1, _NB, _D), lambda h: (h, 0, 0)),
        ],
        out_shape=[
            jax.ShapeDtypeStruct((_H, _NB, _CBS * _D), jnp.float32),
            jax.ShapeDtypeStruct((_H, _NB, _CBS * _D), jnp.float32),
            jax.ShapeDtypeStruct((_H, _NB, _D), jnp.float32),
        ],
    )(k16, v16, pe, comp_w1, cb1, comp_w2, cb2, csum)

    kcf = kc.reshape(_H, _NC, _D)
    vcf = vc.reshape(_H, _NC, _D)
    expand = jnp.repeat(jnp.eye(_NB, dtype=jnp.bfloat16), _CBS, axis=1)

    out = pl.pallas_call(
        _attn_kernel,
        grid=(_H, _NT),
        in_specs=[
            pl.BlockSpec((1, _QT, _D), lambda h, t: (h, t, 0)),
            pl.BlockSpec((1, _S, _D), lambda h, t: (h, 0, 0)),
            pl.BlockSpec((1, _S, _D), lambda h, t: (h, 0, 0)),
            pl.BlockSpec((1, _NC, _D), lambda h, t: (h, 0, 0)),
            pl.BlockSpec((1, _NC, _D), lambda h, t: (h, 0, 0)),
            pl.BlockSpec((1, _NB, _D), lambda h, t: (h, 0, 0)),
            pl.BlockSpec((_D, _GH), lambda h, t: (0, 0)),
            pl.BlockSpec((1, _GH), lambda h, t: (0, 0)),
            pl.BlockSpec((_GH, 3), lambda h, t: (0, 0)),
            pl.BlockSpec((1, 3), lambda h, t: (0, 0)),
            pl.BlockSpec((_NB, _NC), lambda h, t: (0, 0)),
        ],
        out_specs=pl.BlockSpec((1, _QT, _D), lambda h, t: (h, t, 0)),
        out_shape=jax.ShapeDtypeStruct((_H, _S, _D), jnp.float32),
    )(q_t, k_t, v_t, kcf, vcf, ks, gate_w1, gb1, gate_w2, gb2, expand)

    return jnp.transpose(out, (1, 0, 2))[None]


# speedup vs baseline: 27.1924x; 1.0583x over previous
import jax
import jax.numpy as jnp
from jax.experimental import pallas as pl

_B, _S, _H, _D = 1, 2048, 12, 64
_W = 32
_BS = 32
_STRIDE = 16
_CBS = 8
_NSEL = 4
_GH = 128
_CH = 512
_NB = (_S - _BS) // _STRIDE + 1
_NC = _NB * _CBS
_QT = 1024
_NT = _S // _QT
_WT = _QT // _W
_SCALE = 0.125


def _compress_kernel(k16_ref, v16_ref, pe_ref, w1_ref, b1_ref, w2_ref, b2_ref,
                     m_ref, kc_ref, vc_ref, ks_ref):
    k16 = k16_ref[0]
    v16 = v16_ref[0]
    pe = pe_ref[...]
    w1 = w1_ref[...]
    w1a = w1[:_STRIDE * _D].astype(jnp.bfloat16)
    w1b = w1[_STRIDE * _D:].astype(jnp.bfloat16)
    b1 = b1_ref[...]
    w2b = w2_ref[...].astype(jnp.bfloat16)
    b2 = b2_ref[...]

    def mlp(x16, use_pe):
        a = x16[:_NB]
        b = x16[1:]
        if use_pe:
            a = a + pe[:, :_STRIDE * _D]
            b = b + pe[:, _STRIDE * _D:]
        h = jax.nn.gelu(
            jnp.dot(a.astype(jnp.bfloat16), w1a,
                    preferred_element_type=jnp.float32)
            + jnp.dot(b.astype(jnp.bfloat16), w1b,
                      preferred_element_type=jnp.float32) + b1)
        return jnp.dot(h.astype(jnp.bfloat16), w2b,
                       preferred_element_type=jnp.float32) + b2

    kcmp = mlp(k16, True)
    kc_ref[0] = kcmp
    vc_ref[0] = mlp(v16, False)
    ks_ref[0] = jnp.dot(kcmp, m_ref[...],
                        preferred_element_type=jnp.float32,
                        precision=jax.lax.Precision.HIGHEST)


def _attn_kernel(q_ref, k_ref, v_ref, kcf_ref, vcf_ref, ks_ref,
                 gw1_ref, gb1_ref, gw2_ref, gb2_ref, e_ref, o_ref):
    t = pl.program_id(1)
    q = q_ref[0]
    kcf = kcf_ref[0]
    vcfb = vcf_ref[0].astype(jnp.bfloat16)
    vx = jnp.concatenate(
        [vcfb, jnp.ones((_NC, 1), jnp.bfloat16)], axis=1)
    qb = q.astype(jnp.bfloat16)
    qs = (q * _SCALE).astype(jnp.bfloat16)
    ex = e_ref[...]

    gh = jax.nn.gelu(
        jnp.dot(qb, gw1_ref[...].astype(jnp.bfloat16),
                preferred_element_type=jnp.float32) + gb1_ref[...])
    g = jax.nn.sigmoid(
        jnp.dot(gh.astype(jnp.bfloat16), gw2_ref[...].astype(jnp.bfloat16),
                preferred_element_type=jnp.float32) + gb2_ref[...])
    g = g / (jnp.sum(g, axis=1, keepdims=True) + 1e-6)

    sc = jax.lax.dot_general(
        qs, kcf.astype(jnp.bfloat16), (((1,), (1,)), ((), ())),
        preferred_element_type=jnp.float32)

    bsc = jax.lax.dot_general(
        qb, ks_ref[0].astype(jnp.bfloat16), (((1,), (1,)), ((), ())),
        preferred_element_type=jnp.float32) * _SCALE
    brow = jax.lax.broadcasted_iota(jnp.int32, (_QT, _NB), 0) + t * _QT
    bcol = jax.lax.broadcasted_iota(jnp.int32, (_QT, _NB), 1)
    mb = brow >= bcol * _STRIDE
    cm = jnp.where(mb, bsc, -1e9)
    cur = cm
    for _ in range(_NSEL - 1):
        mxb = jnp.max(cur, axis=1, keepdims=True)
        cur = jnp.where(cur >= mxb, -3e38, cur)
    thr = jnp.max(cur, axis=1, keepdims=True)
    mbf = jnp.where(mb, 1.0, 0.0).astype(jnp.bfloat16)
    sbf = jnp.where((cm >= thr) & mb, 1.0, 0.0).astype(jnp.bfloat16)
    m01 = jnp.dot(mbf, ex, preferred_element_type=jnp.float32)
    s01 = jnp.dot(sbf, ex, preferred_element_type=jnp.float32)

    mx = jnp.max(sc, axis=1, keepdims=True)
    e = jnp.exp(sc - mx)
    me = e * m01
    mesel = e * s01
    o_cmp = jnp.dot(me.astype(jnp.bfloat16), vx,
                    preferred_element_type=jnp.float32)
    o_sel = jnp.dot(mesel.astype(jnp.bfloat16), vx,
                    preferred_element_type=jnp.float32)
    s_cmp = o_cmp[:, _D:_D + 1]
    s_sel = o_sel[:, _D:_D + 1]

    pstart = jnp.maximum(_QT * t - _W, 0)

    def window_tiles(x_ref):
        cur = x_ref[0, pl.ds(_QT * t, _QT), :].reshape(_WT, _W, _D)
        prev = x_ref[0, pl.ds(pstart, _QT), :].reshape(_WT, _W, _D)
        prev0 = jnp.concatenate([prev[:1], prev[:_WT - 1]], axis=0)
        return cur, jnp.where(t == 0, prev0, prev)

    kc, kp = window_tiles(k_ref)
    vc, vp = window_tiles(v_ref)
    k2 = jnp.concatenate([kp, kc], axis=1).astype(jnp.bfloat16)
    v2 = jnp.concatenate([vp, vc], axis=1).astype(jnp.bfloat16)
    qw = qs.reshape(_WT, _W, _D)
    scw = jax.lax.dot_general(
        qw, k2, (((2,), (2,)), ((0,), (0,))),
        preferred_element_type=jnp.float32)
    ii = jax.lax.broadcasted_iota(jnp.int32, (_WT, _W, 2 * _W), 1)
    jj = jax.lax.broadcasted_iota(jnp.int32, (_WT, _W, 2 * _W), 2)
    uu = jax.lax.broadcasted_iota(jnp.int32, (_WT, _W, 2 * _W), 0) + t * _WT
    mw = (jj >= ii + 1) & (jj <= ii + _W) & ((uu > 0) | (jj >= _W))
    xm = jnp.where(mw, scw, -1e9)
    mxw = jnp.max(xm, axis=2, keepdims=True)
    ew = jnp.exp(xm - mxw)
    pw = ew / jnp.sum(ew, axis=2, keepdims=True)
    out_win = jax.lax.dot_general(
        pw.astype(jnp.bfloat16), v2, (((2,), (1,)), ((0,), (0,))),
        preferred_element_type=jnp.float32).reshape(_QT, _D)

    o_ref[0] = (o_cmp[:, :_D] * (g[:, 0:1] / s_cmp)
                + o_sel[:, :_D] * (g[:, 1:2] / s_sel)
                + out_win * g[:, 2:3])


def kernel(q, k, v, gate_w1, gate_b1, gate_w2, gate_b2,
           comp_w1, comp_b1, comp_w2, comp_b2, pos_enc):
    q_t = jnp.transpose(q[0], (1, 0, 2))
    k_t = jnp.transpose(k[0], (1, 0, 2))
    v_t = jnp.transpose(v[0], (1, 0, 2))
    k16 = k_t.reshape(_H, _S // _STRIDE, _STRIDE * _D)
    v16 = v_t.reshape(_H, _S // _STRIDE, _STRIDE * _D)
    pe = pos_enc.reshape(1, _BS * _D)
    cb1 = comp_b1.reshape(1, _CH)
    cb2 = comp_b2.reshape(1, _CBS * _D)
    gb1 = gate_b1.reshape(1, _GH)
    gb2 = gate_b2.reshape(1, 3)

    csum = jnp.tile(jnp.eye(_D, dtype=jnp.float32), (_CBS, 1))

    kc, vc, ks = pl.pallas_call(
        _compress_kernel,
        grid=(_H,),
        in_specs=[
            pl.BlockSpec((1, _S // _STRIDE, _STRIDE * _D), lambda h: (h, 0, 0)),
            pl.BlockSpec((1, _S // _STRIDE, _STRIDE * _D), lambda h: (h, 0, 0)),
            pl.BlockSpec((1, _BS * _D), lambda h: (0, 0)),
            pl.BlockSpec((_BS * _D, _CH), lambda h: (0, 0)),
            pl.BlockSpec((1, _CH), lambda h: (0, 0)),
            pl.BlockSpec((_CH, _CBS * _D), lambda h: (0, 0)),
            pl.BlockSpec((1, _CBS * _D), lambda h: (0, 0)),
            pl.BlockSpec((_CBS * _D, _D), lambda h: (0, 0)),
        ],
        out_specs=[
            pl.BlockSpec((1, _NB, _CBS * _D), lambda h: (h, 0, 0)),
            pl.BlockSpec((1, _NB, _CBS * _D), lambda h: (h, 0, 0)),
            pl.BlockSpec((1, _NB, _D), lambda h: (h, 0, 0)),
        ],
        out_shape=[
            jax.ShapeDtypeStruct((_H, _NB, _CBS * _D), jnp.float32),
            jax.ShapeDtypeStruct((_H, _NB, _CBS * _D), jnp.float32),
            jax.ShapeDtypeStruct((_H, _NB, _D), jnp.float32),
        ],
    )(k16, v16, pe, comp_w1, cb1, comp_w2, cb2, csum)

    kcf = kc.reshape(_H, _NC, _D)
    vcf = vc.reshape(_H, _NC, _D)
    expand = jnp.repeat(jnp.eye(_NB, dtype=jnp.bfloat16), _CBS, axis=1)

    out = pl.pallas_call(
        _attn_kernel,
        grid=(_H, _NT),
        in_specs=[
            pl.BlockSpec((1, _QT, _D), lambda h, t: (h, t, 0)),
            pl.BlockSpec((1, _S, _D), lambda h, t: (h, 0, 0)),
            pl.BlockSpec((1, _S, _D), lambda h, t: (h, 0, 0)),
            pl.BlockSpec((1, _NC, _D), lambda h, t: (h, 0, 0)),
            pl.BlockSpec((1, _NC, _D), lambda h, t: (h, 0, 0)),
            pl.BlockSpec((1, _NB, _D), lambda h, t: (h, 0, 0)),
            pl.BlockSpec((_D, _GH), lambda h, t: (0, 0)),
            pl.BlockSpec((1, _GH), lambda h, t: (0, 0)),
            pl.BlockSpec((_GH, 3), lambda h, t: (0, 0)),
            pl.BlockSpec((1, 3), lambda h, t: (0, 0)),
            pl.BlockSpec((_NB, _NC), lambda h, t: (0, 0)),
        ],
        out_specs=pl.BlockSpec((1, _QT, _D), lambda h, t: (h, t, 0)),
        out_shape=jax.ShapeDtypeStruct((_H, _S, _D), jnp.float32),
    )(q_t, k_t, v_t, kcf, vcf, ks, gate_w1, gb1, gate_w2, gb2, expand)

    return jnp.transpose(out, (1, 0, 2))[None]
